# Initial kernel scaffold; baseline (speedup 1.0000x reference)
import jax
import jax.numpy as jnp
from jax.experimental import pallas as pl


def kernel(s, r, o, e_table, r_table):
    raise NotImplementedError("write your pallas kernel here")


# speedup vs baseline: 1.6269x; 1.6269x over previous
import functools

import jax
import jax.numpy as jnp
from jax import lax
from jax.experimental import pallas as pl
from jax.experimental.pallas import tpu as pltpu
from jax.experimental.pallas import tpu_sc as plsc

Z = 32
ROW = 2 * Z
NC, NS = 2, 16
NW = NC * NS
SUB = 128
NSUB = 8
CHUNK = SUB * NSUB

_MESH = dict(core_axis_name="c", subcore_axis_name="s")


def _transform_rows(src_ref, dst_ref, n_rows):
    def body(i, carry):
        for h in range(Z // 16):
            m = src_ref[i, pl.ds(h * 16, 16)]
            lv = src_ref[i, pl.ds(Z + h * 16, 16)]
            dst_ref[i, pl.ds(h * 16, 16)] = jnp.exp(lv * 0.5) + m
        return carry
    lax.fori_loop(0, n_rows, body, 0, unroll=4)


def _make_r_table_kernel(nr_pad):
    rows_per = nr_pad // NW
    mesh = plsc.VectorSubcoreMesh(**_MESH)

    @functools.partial(
        pl.kernel,
        mesh=mesh,
        compiler_params=pltpu.CompilerParams(use_tc_tiling_on_sc=False),
        out_type=jax.ShapeDtypeStruct((nr_pad, Z), jnp.float32),
        scratch_types=[
            pltpu.VMEM((rows_per, ROW), jnp.float32),
            pltpu.VMEM((rows_per, Z), jnp.float32),
        ],
    )
    def k(rtab_hbm, out_hbm, rbuf, obuf):
        wid = lax.axis_index("s") * NC + lax.axis_index("c")
        base = wid * rows_per
        pltpu.sync_copy(rtab_hbm.at[pl.ds(base, rows_per)], rbuf)
        _transform_rows(rbuf, obuf, rows_per)
        pltpu.sync_copy(obuf, out_hbm.at[pl.ds(base, rows_per)])

    return k


def _make_main_kernel(total, nr_pad):
    per_w = total // NW
    n_chunks = per_w // CHUNK
    n_idx_rows = per_w // SUB
    mesh = plsc.VectorSubcoreMesh(**_MESH)
    out_sds = jax.ShapeDtypeStruct((total, Z), jnp.float32)

    @functools.partial(
        pl.kernel,
        mesh=mesh,
        compiler_params=pltpu.CompilerParams(use_tc_tiling_on_sc=False),
        out_type=(out_sds, out_sds, out_sds),
        scratch_types=[
            pltpu.VMEM((NSUB, SUB), jnp.int32),
            pltpu.VMEM((CHUNK, ROW), jnp.float32),
            pltpu.VMEM((CHUNK, Z), jnp.float32),
            pltpu.SemaphoreType.DMA,
        ],
    )
    def k(s_hbm, o_hbm, r_hbm, etab_hbm, zrtab_hbm,
          zs_hbm, zo_hbm, zr_hbm, idx_v, rows_v, out_v, sem):
        wid = lax.axis_index("s") * NC + lax.axis_index("c")
        base = wid * per_w
        idx_base = wid * (per_w // SUB)

        def e_chunk(idx_hbm, out_hbm, c, carry):
            off = base + c * CHUNK
            pltpu.sync_copy(idx_hbm.at[pl.ds(idx_base + c * NSUB, NSUB)],
                            idx_v)
            cps = [
                pltpu.async_copy(etab_hbm.at[idx_v.at[j]],
                                 rows_v.at[pl.ds(j * SUB, SUB)], sem)
                for j in range(NSUB)
            ]
            for cp in cps:
                cp.wait()
            _transform_rows(rows_v, out_v, CHUNK)
            pltpu.sync_copy(out_v, out_hbm.at[pl.ds(off, CHUNK)])
            return carry

        def r_chunk(c, carry):
            off = base + c * CHUNK
            pltpu.sync_copy(r_hbm.at[pl.ds(idx_base + c * NSUB, NSUB)], idx_v)
            cps = [
                pltpu.async_copy(zrtab_hbm.at[idx_v.at[j]],
                                 out_v.at[pl.ds(j * SUB, SUB)], sem)
                for j in range(NSUB)
            ]
            for cp in cps:
                cp.wait()
            pltpu.sync_copy(out_v, zr_hbm.at[pl.ds(off, CHUNK)])
            return carry

        lax.fori_loop(0, n_chunks, functools.partial(e_chunk, s_hbm, zs_hbm),
                      0)
        lax.fori_loop(0, n_chunks, functools.partial(e_chunk, o_hbm, zo_hbm),
                      0)
        lax.fori_loop(0, n_chunks, r_chunk, 0)

    return k


def kernel(s, r, o, e_table, r_table):
    b, l = s.shape
    total = b * l
    nr = r_table.shape[0]
    nr_pad = ((nr + NW - 1) // NW) * NW
    if nr_pad % 8:
        nr_pad += 8 - nr_pad % 8

    s_idx = s.reshape(total // SUB, SUB).astype(jnp.int32)
    o_idx = o.reshape(total // SUB, SUB).astype(jnp.int32)
    r_idx = r.reshape(total // SUB, SUB).astype(jnp.int32)

    r_pad = jnp.zeros((nr_pad, ROW), jnp.float32).at[:nr].set(r_table)
    zr_table = _make_r_table_kernel(nr_pad)(r_pad)

    zs, zo, zr = _make_main_kernel(total, nr_pad)(
        s_idx, o_idx, r_idx, e_table, zr_table)
    return (zs.reshape(b, l, Z), zr.reshape(b, l, Z), zo.reshape(b, l, Z))


# speedup vs baseline: 1.6671x; 1.0247x over previous
import functools

import jax
import jax.numpy as jnp
from jax import lax
from jax.experimental import pallas as pl
from jax.experimental.pallas import tpu as pltpu
from jax.experimental.pallas import tpu_sc as plsc

Z = 32
ROW = 2 * Z
NC, NS = 2, 16
NW = NC * NS
SUB = 128
NSUB = 4
CHUNK = SUB * NSUB

_MESH = dict(core_axis_name="c", subcore_axis_name="s")
_NO_TC_TILING = pltpu.CompilerParams(use_tc_tiling_on_sc=False)


def _transform_rows(src_ref, dst_ref, n_rows):
    def body(i, carry):
        for h in range(Z // 16):
            m = src_ref[i, pl.ds(h * 16, 16)]
            lv = src_ref[i, pl.ds(Z + h * 16, 16)]
            dst_ref[i, pl.ds(h * 16, 16)] = jnp.exp(lv * 0.5) + m
        return carry
    lax.fori_loop(0, n_rows, body, 0, unroll=4)


def _make_r_table_kernel(nr_pad):
    rows_per = nr_pad // NW
    mesh = plsc.VectorSubcoreMesh(**_MESH)

    @functools.partial(
        pl.kernel,
        mesh=mesh,
        compiler_params=_NO_TC_TILING,
        out_type=jax.ShapeDtypeStruct((nr_pad, Z), jnp.float32),
        scratch_types=[
            pltpu.VMEM((rows_per, ROW), jnp.float32),
            pltpu.VMEM((rows_per, Z), jnp.float32),
        ],
    )
    def k(rtab_hbm, out_hbm, rbuf, obuf):
        wid = lax.axis_index("s") * NC + lax.axis_index("c")
        base = wid * rows_per
        pltpu.sync_copy(rtab_hbm.at[pl.ds(base, rows_per)], rbuf)
        _transform_rows(rbuf, obuf, rows_per)
        pltpu.sync_copy(obuf, out_hbm.at[pl.ds(base, rows_per)])

    return k


def _make_main_kernel(total, nr_pad):
    per_w = total // NW
    n_chunks = per_w // CHUNK
    mesh = plsc.VectorSubcoreMesh(**_MESH)
    out_sds = jax.ShapeDtypeStruct((total, Z), jnp.float32)

    @functools.partial(
        pl.kernel,
        mesh=mesh,
        compiler_params=_NO_TC_TILING,
        out_type=(out_sds, out_sds, out_sds),
        scratch_types=[
            pltpu.VMEM((2, NSUB, SUB), jnp.int32),
            pltpu.VMEM((2, CHUNK, ROW), jnp.float32),
            pltpu.VMEM((2, CHUNK, Z), jnp.float32),
            pltpu.SemaphoreType.DMA,
            pltpu.SemaphoreType.DMA,
            pltpu.SemaphoreType.DMA,
            pltpu.SemaphoreType.DMA,
        ],
    )
    def k(s_hbm, o_hbm, r_hbm, etab_hbm, zrtab_hbm,
          zs_hbm, zo_hbm, zr_hbm, idx_v, rows_v, out_v,
          gsem0, gsem1, osem0, osem1):
        wid = lax.axis_index("s") * NC + lax.axis_index("c")
        base = wid * per_w
        idx_base = wid * (per_w // SUB)
        gsems = (gsem0, gsem1)
        osems = (osem0, osem1)

        sched = ([("e", s_hbm, zs_hbm, c) for c in range(n_chunks)]
                 + [("e", o_hbm, zo_hbm, c) for c in range(n_chunks)]
                 + [("r", r_hbm, zr_hbm, c) for c in range(n_chunks)])
        n = len(sched)

        def fire(i, p):
            kind, idx_hbm, _, c = sched[i]
            pltpu.sync_copy(idx_hbm.at[pl.ds(idx_base + c * NSUB, NSUB)],
                            idx_v.at[p])
            if kind == "e":
                return [
                    pltpu.async_copy(
                        etab_hbm.at[idx_v.at[p, j]],
                        rows_v.at[p, pl.ds(j * SUB, SUB)], gsems[p])
                    for j in range(NSUB)
                ]
            return [
                pltpu.async_copy(
                    zrtab_hbm.at[idx_v.at[p, j]],
                    out_v.at[p, pl.ds(j * SUB, SUB)], gsems[p])
                for j in range(NSUB)
            ]

        pending_g = fire(0, 0)
        pending_o = [None, None]
        for i in range(n):
            p = i % 2
            q = 1 - p
            kind, _, out_hbm, c = sched[i]
            for cp in pending_g:
                cp.wait()
            if i + 1 < n:
                if pending_o[q] is not None:
                    pending_o[q].wait()
                    pending_o[q] = None
                pending_g = fire(i + 1, q)
            if kind == "e":
                _transform_rows(rows_v.at[p], out_v.at[p], CHUNK)
            off = base + c * CHUNK
            pending_o[p] = pltpu.async_copy(
                out_v.at[p], out_hbm.at[pl.ds(off, CHUNK)], osems[p])
        for po in pending_o:
            if po is not None:
                po.wait()

    return k


def kernel(s, r, o, e_table, r_table):
    b, l = s.shape
    total = b * l
    nr = r_table.shape[0]
    nr_pad = ((nr + NW - 1) // NW) * NW
    if nr_pad % 8:
        nr_pad += 8 - nr_pad % 8

    s_idx = s.reshape(total // SUB, SUB).astype(jnp.int32)
    o_idx = o.reshape(total // SUB, SUB).astype(jnp.int32)
    r_idx = r.reshape(total // SUB, SUB).astype(jnp.int32)

    r_pad = jnp.zeros((nr_pad, ROW), jnp.float32).at[:nr].set(r_table)
    zr_table = _make_r_table_kernel(nr_pad)(r_pad)

    zs, zo, zr = _make_main_kernel(total, nr_pad)(
        s_idx, o_idx, r_idx, e_table, zr_table)
    return (zs.reshape(b, l, Z), zr.reshape(b, l, Z), zo.reshape(b, l, Z))


# speedup vs baseline: 2.2443x; 1.3462x over previous
import functools

import jax
import jax.numpy as jnp
from jax import lax
from jax.experimental import pallas as pl
from jax.experimental.pallas import tpu as pltpu
from jax.experimental.pallas import tpu_sc as plsc

Z = 32
ROW = 2 * Z
NC, NS = 2, 16
NW = NC * NS
SUB = 128
NSUB = 4
CHUNK = SUB * NSUB

_MESH = dict(core_axis_name="c", subcore_axis_name="s")
_NO_TC_TILING = pltpu.CompilerParams(use_tc_tiling_on_sc=False)


def _transform_rows(src_ref, dst_ref, n_rows):
    @plsc.parallel_loop(0, n_rows, unroll=4)
    def body(i):
        for h in range(Z // 16):
            m = src_ref[i, pl.ds(h * 16, 16)]
            lv = src_ref[i, pl.ds(Z + h * 16, 16)]
            dst_ref[i, pl.ds(h * 16, 16)] = jnp.exp(lv * 0.5) + m


def _make_r_table_kernel(nr_pad):
    rows_per = nr_pad // NW
    mesh = plsc.VectorSubcoreMesh(**_MESH)

    @functools.partial(
        pl.kernel,
        mesh=mesh,
        compiler_params=_NO_TC_TILING,
        out_type=jax.ShapeDtypeStruct((nr_pad, Z), jnp.float32),
        scratch_types=[
            pltpu.VMEM((rows_per, ROW), jnp.float32),
            pltpu.VMEM((rows_per, Z), jnp.float32),
        ],
    )
    def k(rtab_hbm, out_hbm, rbuf, obuf):
        wid = lax.axis_index("s") * NC + lax.axis_index("c")
        base = wid * rows_per
        pltpu.sync_copy(rtab_hbm.at[pl.ds(base, rows_per)], rbuf)
        _transform_rows(rbuf, obuf, rows_per)
        pltpu.sync_copy(obuf, out_hbm.at[pl.ds(base, rows_per)])

    return k


def _make_main_kernel(total, nr_pad):
    per_w = total // NW
    n_chunks = per_w // CHUNK
    mesh = plsc.VectorSubcoreMesh(**_MESH)
    out_sds = jax.ShapeDtypeStruct((total, Z), jnp.float32)

    @functools.partial(
        pl.kernel,
        mesh=mesh,
        compiler_params=_NO_TC_TILING,
        out_type=(out_sds, out_sds, out_sds),
        scratch_types=[
            pltpu.VMEM((2, NSUB, SUB), jnp.int32),
            pltpu.VMEM((2, CHUNK, ROW), jnp.float32),
            pltpu.VMEM((2, CHUNK, Z), jnp.float32),
            pltpu.SemaphoreType.DMA,
            pltpu.SemaphoreType.DMA,
            pltpu.SemaphoreType.DMA,
            pltpu.SemaphoreType.DMA,
        ],
    )
    def k(s_hbm, o_hbm, r_hbm, etab_hbm, zrtab_hbm,
          zs_hbm, zo_hbm, zr_hbm, idx_v, rows_v, out_v,
          gsem0, gsem1, osem0, osem1):
        wid = lax.axis_index("s") * NC + lax.axis_index("c")
        base = wid * per_w
        idx_base = wid * (per_w // SUB)
        gsems = (gsem0, gsem1)
        osems = (osem0, osem1)

        sched = ([("e", s_hbm, zs_hbm, c) for c in range(n_chunks)]
                 + [("e", o_hbm, zo_hbm, c) for c in range(n_chunks)]
                 + [("r", r_hbm, zr_hbm, c) for c in range(n_chunks)])
        n = len(sched)

        def fire(i, p):
            kind, idx_hbm, _, c = sched[i]
            pltpu.sync_copy(idx_hbm.at[pl.ds(idx_base + c * NSUB, NSUB)],
                            idx_v.at[p])
            if kind == "e":
                return [
                    pltpu.async_copy(
                        etab_hbm.at[idx_v.at[p, j]],
                        rows_v.at[p, pl.ds(j * SUB, SUB)], gsems[p])
                    for j in range(NSUB)
                ]
            return [
                pltpu.async_copy(
                    zrtab_hbm.at[idx_v.at[p, j]],
                    out_v.at[p, pl.ds(j * SUB, SUB)], gsems[p])
                for j in range(NSUB)
            ]

        pending_g = fire(0, 0)
        pending_o = [None, None]
        for i in range(n):
            p = i % 2
            q = 1 - p
            kind, _, out_hbm, c = sched[i]
            for cp in pending_g:
                cp.wait()
            if i + 1 < n:
                if pending_o[q] is not None:
                    pending_o[q].wait()
                    pending_o[q] = None
                pending_g = fire(i + 1, q)
            if kind == "e":
                _transform_rows(rows_v.at[p], out_v.at[p], CHUNK)
            off = base + c * CHUNK
            pending_o[p] = pltpu.async_copy(
                out_v.at[p], out_hbm.at[pl.ds(off, CHUNK)], osems[p])
        for po in pending_o:
            if po is not None:
                po.wait()

    return k


def kernel(s, r, o, e_table, r_table):
    b, l = s.shape
    total = b * l
    nr = r_table.shape[0]
    nr_pad = ((nr + NW - 1) // NW) * NW
    if nr_pad % 8:
        nr_pad += 8 - nr_pad % 8

    s_idx = s.reshape(total // SUB, SUB).astype(jnp.int32)
    o_idx = o.reshape(total // SUB, SUB).astype(jnp.int32)
    r_idx = r.reshape(total // SUB, SUB).astype(jnp.int32)

    r_pad = jnp.zeros((nr_pad, ROW), jnp.float32).at[:nr].set(r_table)
    zr_table = _make_r_table_kernel(nr_pad)(r_pad)

    zs, zo, zr = _make_main_kernel(total, nr_pad)(
        s_idx, o_idx, r_idx, e_table, zr_table)
    return (zs.reshape(b, l, Z), zr.reshape(b, l, Z), zo.reshape(b, l, Z))


# speedup vs baseline: 2.2585x; 1.0063x over previous
import functools

import jax
import jax.numpy as jnp
from jax import lax
from jax.experimental import pallas as pl
from jax.experimental.pallas import tpu as pltpu
from jax.experimental.pallas import tpu_sc as plsc

Z = 32
ROW = 2 * Z
NC, NS = 2, 16
NW = NC * NS
SUB = 128
NSUB = 4
CHUNK = SUB * NSUB
NBUF = 2

_MESH = dict(core_axis_name="c", subcore_axis_name="s")
_NO_TC_TILING = pltpu.CompilerParams(use_tc_tiling_on_sc=False)


def _transform_rows(src_ref, dst_ref, n_rows):
    @plsc.parallel_loop(0, n_rows, unroll=4)
    def body(i):
        for h in range(Z // 16):
            m = src_ref[i, pl.ds(h * 16, 16)]
            lv = src_ref[i, pl.ds(Z + h * 16, 16)]
            dst_ref[i, pl.ds(h * 16, 16)] = jnp.exp(lv * 0.5) + m


def _make_r_table_kernel(nr_pad):
    rows_per = nr_pad // NW
    mesh = plsc.VectorSubcoreMesh(**_MESH)

    @functools.partial(
        pl.kernel,
        mesh=mesh,
        compiler_params=_NO_TC_TILING,
        out_type=jax.ShapeDtypeStruct((nr_pad, Z), jnp.float32),
        scratch_types=[
            pltpu.VMEM((rows_per, ROW), jnp.float32),
            pltpu.VMEM((rows_per, Z), jnp.float32),
        ],
    )
    def k(rtab_hbm, out_hbm, rbuf, obuf):
        wid = lax.axis_index("s") * NC + lax.axis_index("c")
        base = wid * rows_per
        pltpu.sync_copy(rtab_hbm.at[pl.ds(base, rows_per)], rbuf)
        _transform_rows(rbuf, obuf, rows_per)
        pltpu.sync_copy(obuf, out_hbm.at[pl.ds(base, rows_per)])

    return k


def _make_gather_kernel(total):
    per_w = total // NW
    n_chunks = per_w // CHUNK
    mesh = plsc.VectorSubcoreMesh(**_MESH)
    out_sds = jax.ShapeDtypeStruct((total, Z), jnp.float32)

    @functools.partial(
        pl.kernel,
        mesh=mesh,
        compiler_params=_NO_TC_TILING,
        out_type=(out_sds, out_sds, out_sds),
        scratch_types=[
            pltpu.VMEM((NBUF, NSUB, SUB), jnp.int32),
            pltpu.VMEM((NBUF, CHUNK, ROW), jnp.float32),
            pltpu.VMEM((NBUF, CHUNK, Z), jnp.float32),
        ] + [pltpu.SemaphoreType.DMA] * (2 * NBUF),
    )
    def k(s_hbm, o_hbm, r_hbm, e_hbm, zr_hbm,
          zs_hbm, zo_hbm, zr_out_hbm, idx_v, ebuf_v, obuf_v, *sems):
        gsems = sems[:NBUF]
        osems = sems[NBUF:]
        wid = lax.axis_index("s") * NC + lax.axis_index("c")
        base = wid * per_w
        idx_base = wid * (per_w // SUB)

        sched = ([(s_hbm, e_hbm, zs_hbm, True, c) for c in range(n_chunks)]
                 + [(o_hbm, e_hbm, zo_hbm, True, c) for c in range(n_chunks)]
                 + [(r_hbm, zr_hbm, zr_out_hbm, False, c)
                    for c in range(n_chunks)])
        n = len(sched)

        pending_g = [None] * NBUF
        pending_o = [None] * NBUF

        def prep(i):
            p = i % NBUF
            idx_hbm, tab_hbm, _, is_e, c = sched[i]
            if pending_o[p] is not None:
                pending_o[p].wait()
                pending_o[p] = None
            pltpu.sync_copy(idx_hbm.at[pl.ds(idx_base + c * NSUB, NSUB)],
                            idx_v.at[p])
            dst = ebuf_v if is_e else obuf_v
            pending_g[p] = [
                pltpu.async_copy(tab_hbm.at[idx_v.at[p, j]],
                                 dst.at[p, pl.ds(j * SUB, SUB)], gsems[p])
                for j in range(NSUB)
            ]

        def complete(i):
            p = i % NBUF
            _, _, out_hbm, is_e, c = sched[i]
            for cp in pending_g[p]:
                cp.wait()
            pending_g[p] = None
            if is_e:
                _transform_rows(ebuf_v.at[p], obuf_v.at[p], CHUNK)
            pending_o[p] = pltpu.async_copy(
                obuf_v.at[p], out_hbm.at[pl.ds(base + c * CHUNK, CHUNK)],
                osems[p])

        depth = NBUF - 1
        for i in range(min(depth, n)):
            prep(i)
        for i in range(n):
            if i + depth < n:
                prep(i + depth)
            complete(i)
        for po in pending_o:
            if po is not None:
                po.wait()

    return k


def kernel(s, r, o, e_table, r_table):
    b, l = s.shape
    total = b * l
    nr = r_table.shape[0]
    nr_pad = ((nr + NW - 1) // NW) * NW
    if nr_pad % 8:
        nr_pad += 8 - nr_pad % 8

    s_idx = s.reshape(total // SUB, SUB).astype(jnp.int32)
    o_idx = o.reshape(total // SUB, SUB).astype(jnp.int32)
    r_idx = r.reshape(total // SUB, SUB).astype(jnp.int32)

    r_pad = jnp.zeros((nr_pad, ROW), jnp.float32).at[:nr].set(r_table)
    zr_table = _make_r_table_kernel(nr_pad)(r_pad)

    zs, zo, zr = _make_gather_kernel(total)(
        s_idx, o_idx, r_idx, e_table, zr_table)
    return (zs.reshape(b, l, Z), zr.reshape(b, l, Z), zo.reshape(b, l, Z))


# speedup vs baseline: 2.3923x; 1.0592x over previous
import functools

import jax
import jax.numpy as jnp
from jax import lax
from jax.experimental import pallas as pl
from jax.experimental.pallas import tpu as pltpu
from jax.experimental.pallas import tpu_sc as plsc

Z = 32
ROW = 2 * Z
NC, NS = 2, 16
NW = NC * NS
SUB = 128
NSUB = 4
CHUNK = SUB * NSUB
PACK = 128 // Z
NBUF = 2

_MESH = dict(core_axis_name="c", subcore_axis_name="s")
_NO_TC_TILING = pltpu.CompilerParams(use_tc_tiling_on_sc=False)


def _transform_pack(src_ref, dst_ref):
    @plsc.parallel_loop(0, CHUNK // PACK, unroll=2)
    def body(j):
        for k in range(PACK):
            for h in range(Z // 16):
                m = src_ref[j * PACK + k, pl.ds(h * 16, 16)]
                lv = src_ref[j * PACK + k, pl.ds(Z + h * 16, 16)]
                dst_ref[j, pl.ds(k * Z + h * 16, 16)] = jnp.exp(lv * 0.5) + m


def _make_stream_kernel(total):
    per_w = total // NW
    n_chunks = per_w // CHUNK
    mesh = plsc.VectorSubcoreMesh(**_MESH)

    @functools.partial(
        pl.kernel,
        mesh=mesh,
        compiler_params=_NO_TC_TILING,
        out_type=jax.ShapeDtypeStruct((total // PACK, 128), jnp.float32),
        scratch_types=[
            pltpu.VMEM((NBUF, NSUB, SUB), jnp.int32),
            pltpu.VMEM((NBUF, CHUNK, ROW), jnp.float32),
            pltpu.VMEM((NBUF, CHUNK // PACK, 128), jnp.float32),
        ] + [pltpu.SemaphoreType.DMA] * (2 * NBUF),
    )
    def k(idx_hbm, tab_hbm, out_hbm, idx_v, ebuf_v, obuf_v, *sems):
        gsems = sems[:NBUF]
        osems = sems[NBUF:]
        wid = lax.axis_index("s") * NC + lax.axis_index("c")
        idx_base = wid * (per_w // SUB)
        out_base = wid * (per_w // PACK)

        pending_g = [None] * NBUF
        pending_o = [None] * NBUF

        def prep(c):
            p = c % NBUF
            if pending_o[p] is not None:
                pending_o[p].wait()
                pending_o[p] = None
            pltpu.sync_copy(idx_hbm.at[pl.ds(idx_base + c * NSUB, NSUB)],
                            idx_v.at[p])
            pending_g[p] = [
                pltpu.async_copy(tab_hbm.at[idx_v.at[p, j]],
                                 ebuf_v.at[p, pl.ds(j * SUB, SUB)], gsems[p])
                for j in range(NSUB)
            ]

        def complete(c):
            p = c % NBUF
            for cp in pending_g[p]:
                cp.wait()
            pending_g[p] = None
            _transform_pack(ebuf_v.at[p], obuf_v.at[p])
            pending_o[p] = pltpu.async_copy(
                obuf_v.at[p],
                out_hbm.at[pl.ds(out_base + c * (CHUNK // PACK),
                                 CHUNK // PACK)],
                osems[p])

        depth = NBUF - 1
        for c in range(min(depth, n_chunks)):
            prep(c)
        for c in range(n_chunks):
            if c + depth < n_chunks:
                prep(c + depth)
            complete(c)
        for po in pending_o:
            if po is not None:
                po.wait()

    return k


def kernel(s, r, o, e_table, r_table):
    b, l = s.shape
    total = b * l

    s_idx = s.reshape(total // SUB, SUB).astype(jnp.int32)
    o_idx = o.reshape(total // SUB, SUB).astype(jnp.int32)
    r_idx = r.reshape(total // SUB, SUB).astype(jnp.int32)

    gk = _make_stream_kernel(total)
    zs = gk(s_idx, e_table)
    zo = gk(o_idx, e_table)
    zr = gk(r_idx, r_table)
    return (zs.reshape(b, l, Z), zr.reshape(b, l, Z), zo.reshape(b, l, Z))


# speedup vs baseline: 2.3955x; 1.0013x over previous
import functools

import jax
import jax.numpy as jnp
from jax import lax
from jax.experimental import pallas as pl
from jax.experimental.pallas import tpu as pltpu
from jax.experimental.pallas import tpu_sc as plsc

Z = 32
ROW = 2 * Z
NC, NS = 2, 16
NW = NC * NS
SUB = 128
NSUB = 5
CHUNK = SUB * NSUB
NBUF = 2

_MESH = dict(core_axis_name="c", subcore_axis_name="s")
_NO_TC_TILING = pltpu.CompilerParams(use_tc_tiling_on_sc=False)


def _transform3d(src_ref, dst_ref, l):
    @plsc.parallel_loop(0, CHUNK, unroll=4)
    def body(i):
        for h in range(Z // 16):
            m = src_ref[i, pl.ds(h * 16, 16)]
            lv = src_ref[i, pl.ds(Z + h * 16, 16)]
            dst_ref[i // l, i % l, pl.ds(h * 16, 16)] = jnp.exp(lv * 0.5) + m


def _make_stream_kernel(b, l):
    total = b * l
    per_w = total // NW
    n_chunks = per_w // CHUNK
    b_chunk = CHUNK // l
    mesh = plsc.VectorSubcoreMesh(**_MESH)

    @functools.partial(
        pl.kernel,
        mesh=mesh,
        compiler_params=_NO_TC_TILING,
        out_type=jax.ShapeDtypeStruct((b, l, Z), jnp.float32),
        scratch_types=[
            pltpu.VMEM((NBUF, NSUB, SUB), jnp.int32),
            pltpu.VMEM((NBUF, CHUNK, ROW), jnp.float32),
            pltpu.VMEM((NBUF, b_chunk, l, Z), jnp.float32),
        ] + [pltpu.SemaphoreType.DMA] * (2 * NBUF),
    )
    def k(idx_hbm, tab_hbm, out_hbm, idx_v, ebuf_v, obuf_v, *sems):
        gsems = sems[:NBUF]
        osems = sems[NBUF:]
        wid = lax.axis_index("s") * NC + lax.axis_index("c")
        idx_base = wid * (per_w // SUB)
        out_base = wid * (per_w // l)

        pending_g = [None] * NBUF
        pending_o = [None] * NBUF

        def prep(c):
            p = c % NBUF
            if pending_o[p] is not None:
                pending_o[p].wait()
                pending_o[p] = None
            pltpu.sync_copy(idx_hbm.at[pl.ds(idx_base + c * NSUB, NSUB)],
                            idx_v.at[p])
            pending_g[p] = [
                pltpu.async_copy(tab_hbm.at[idx_v.at[p, j]],
                                 ebuf_v.at[p, pl.ds(j * SUB, SUB)], gsems[p])
                for j in range(NSUB)
            ]

        def complete(c):
            p = c % NBUF
            for cp in pending_g[p]:
                cp.wait()
            pending_g[p] = None
            _transform3d(ebuf_v.at[p], obuf_v.at[p], l)
            pending_o[p] = pltpu.async_copy(
                obuf_v.at[p],
                out_hbm.at[pl.ds(out_base + c * b_chunk, b_chunk)],
                osems[p])

        depth = NBUF - 1
        for c in range(min(depth, n_chunks)):
            prep(c)
        for c in range(n_chunks):
            if c + depth < n_chunks:
                prep(c + depth)
            complete(c)
        for po in pending_o:
            if po is not None:
                po.wait()

    return k


def kernel(s, r, o, e_table, r_table):
    b, l = s.shape
    total = b * l

    s_idx = s.reshape(total // SUB, SUB).astype(jnp.int32)
    o_idx = o.reshape(total // SUB, SUB).astype(jnp.int32)
    r_idx = r.reshape(total // SUB, SUB).astype(jnp.int32)

    gk = _make_stream_kernel(b, l)
    zs = gk(s_idx, e_table)
    zo = gk(o_idx, e_table)
    zr = gk(r_idx, r_table)
    return (zs, zr, zo)


# speedup vs baseline: 2.8564x; 1.1924x over previous
import functools

import jax
import jax.numpy as jnp
from jax import lax
from jax.experimental import pallas as pl
from jax.experimental.pallas import tpu as pltpu
from jax.experimental.pallas import tpu_sc as plsc

Z = 32
ROW = 2 * Z
NC, NS = 2, 16
NW = NC * NS
SUB = 128
NSUB = 5
CHUNK = SUB * NSUB
NBUF = 2

_MESH = dict(core_axis_name="c", subcore_axis_name="s")
_NO_TC_TILING = pltpu.CompilerParams(use_tc_tiling_on_sc=False)


def _transform3d(src_ref, dst_ref, l):
    @plsc.parallel_loop(0, CHUNK, unroll=4)
    def body(i):
        for h in range(Z // 16):
            m = src_ref[i, pl.ds(h * 16, 16)]
            lv = src_ref[i, pl.ds(Z + h * 16, 16)]
            dst_ref[i // l, i % l, pl.ds(h * 16, 16)] = jnp.exp(lv * 0.5) + m


def _make_stream_kernel(b, l):
    total = b * l
    per_w = total // NW
    n_chunks = per_w // CHUNK
    b_chunk = CHUNK // l
    lpad = ((l + 7) // 8) * 8
    mesh = plsc.VectorSubcoreMesh(**_MESH)

    @functools.partial(
        pl.kernel,
        mesh=mesh,
        compiler_params=_NO_TC_TILING,
        out_type=jax.ShapeDtypeStruct((b * lpad, 128), jnp.float32),
        scratch_types=[
            pltpu.VMEM((NBUF, NSUB, SUB), jnp.int32),
            pltpu.VMEM((NBUF, CHUNK, ROW), jnp.float32),
            pltpu.VMEM((NBUF, b_chunk, l, Z), jnp.float32),
        ] + [pltpu.SemaphoreType.DMA] * (2 * NBUF),
    )
    def k(idx_hbm, tab_hbm, out_hbm, idx_v, ebuf_v, obuf_v, *sems):
        gsems = sems[:NBUF]
        osems = sems[NBUF:]
        wid = lax.axis_index("s") * NC + lax.axis_index("c")
        idx_base = wid * (per_w // SUB)
        out_base = wid * (per_w // l) * lpad

        pending_g = [None] * NBUF
        pending_o = [None] * NBUF

        def prep(c):
            p = c % NBUF
            if pending_o[p] is not None:
                for cp in pending_o[p]:
                    cp.wait()
                pending_o[p] = None
            pltpu.sync_copy(idx_hbm.at[pl.ds(idx_base + c * NSUB, NSUB)],
                            idx_v.at[p])
            pending_g[p] = [
                pltpu.async_copy(tab_hbm.at[idx_v.at[p, j]],
                                 ebuf_v.at[p, pl.ds(j * SUB, SUB)], gsems[p])
                for j in range(NSUB)
            ]

        def complete(c):
            p = c % NBUF
            for cp in pending_g[p]:
                cp.wait()
            pending_g[p] = None
            _transform3d(ebuf_v.at[p], obuf_v.at[p], l)
            pending_o[p] = [
                pltpu.async_copy(
                    obuf_v.at[p, bb],
                    out_hbm.at[pl.ds(out_base + (c * b_chunk + bb) * lpad, l),
                               pl.ds(0, Z)],
                    osems[p])
                for bb in range(b_chunk)
            ]

        depth = NBUF - 1
        for c in range(min(depth, n_chunks)):
            prep(c)
        for c in range(n_chunks):
            if c + depth < n_chunks:
                prep(c + depth)
            complete(c)
        for po in pending_o:
            if po is not None:
                for cp in po:
                    cp.wait()

    return k


def kernel(s, r, o, e_table, r_table):
    b, l = s.shape
    total = b * l

    s_idx = s.reshape(total // SUB, SUB).astype(jnp.int32)
    o_idx = o.reshape(total // SUB, SUB).astype(jnp.int32)
    r_idx = r.reshape(total // SUB, SUB).astype(jnp.int32)

    lpad = ((l + 7) // 8) * 8
    gk = _make_stream_kernel(b, l)

    def run(idx, tab):
        out2 = gk(idx, tab)
        return out2.reshape(b, lpad, 128)[:, :l, :Z]

    zs = run(s_idx, e_table)
    zo = run(o_idx, e_table)
    zr = run(r_idx, r_table)
    return (zs, zr, zo)
